# baseline (speedup 1.0000x reference)
import functools

import jax
import jax.numpy as jnp
from jax import lax
from jax.experimental import pallas as pl
from jax.experimental.pallas import tpu as pltpu
from jax.experimental.pallas import tpu_sc as plsc

F32 = jnp.float32
BF = jnp.bfloat16
I32 = jnp.int32
HID = 128

import numpy as _np
PI = _np.zeros(HID, _np.int64)
SIG = _np.zeros(HID, _np.int64)
for _l in range(4):
    for _i in range(16):
        PI[32 * _l + 2 * _i] = 32 * _l + _i
        PI[32 * _l + 2 * _i + 1] = 32 * _l + 16 + _i
    for _j in range(32):
        SIG[32 * _l + _j] = 32 * _l + (2 * _j if _j < 16
                                       else 2 * (_j - 16) + 1)
NC, NS = 2, 16
NW = NC * NS
CH = 128
NBLK = 1024
EBLK = 2560


def _cdiv(a, b):
    return (a + b - 1) // b


def _mlp2(x, W1, b1, W2, b2, blk, out_relu, out_dtype=F32):
    R, cin = x.shape
    cout = W2.shape[1]

    def body(x_ref, w1_ref, b1_ref, w2_ref, b2_ref, o_ref):
        t = jnp.dot(x_ref[...], w1_ref[...], preferred_element_type=F32)
        t = jnp.maximum(t + b1_ref[...], 0.0)
        y = jnp.dot(t, w2_ref[...], preferred_element_type=F32) + b2_ref[...]
        if out_relu:
            y = jnp.maximum(y, 0.0)
        o_ref[...] = y.astype(out_dtype)

    return pl.pallas_call(
        body,
        grid=(R // blk,),
        in_specs=[
            pl.BlockSpec((blk, cin), lambda i: (i, 0)),
            pl.BlockSpec((cin, HID), lambda i: (0, 0)),
            pl.BlockSpec((1, HID), lambda i: (0, 0)),
            pl.BlockSpec((HID, cout), lambda i: (0, 0)),
            pl.BlockSpec((1, cout), lambda i: (0, 0)),
        ],
        out_specs=pl.BlockSpec((blk, cout), lambda i: (i, 0)),
        out_shape=jax.ShapeDtypeStruct((R, cout), out_dtype),
        compiler_params=pltpu.CompilerParams(
            dimension_semantics=("arbitrary",)),
    )(x, W1, b1.reshape(1, -1), W2, b2.reshape(1, -1))


def _hs_hd(h, Ws, Wd, blk):
    R = h.shape[0]

    def body(h_ref, ws_ref, wd_ref, o1_ref, o2_ref):
        hv = h_ref[...]
        o1_ref[...] = jnp.dot(hv, ws_ref[...], preferred_element_type=F32)
        o2_ref[...] = jnp.dot(hv, wd_ref[...], preferred_element_type=F32)

    return pl.pallas_call(
        body,
        grid=(R // blk,),
        in_specs=[
            pl.BlockSpec((blk, HID), lambda i: (i, 0)),
            pl.BlockSpec((HID, HID), lambda i: (0, 0)),
            pl.BlockSpec((HID, HID), lambda i: (0, 0)),
        ],
        out_specs=[
            pl.BlockSpec((blk, HID), lambda i: (i, 0)),
            pl.BlockSpec((blk, HID), lambda i: (i, 0)),
        ],
        out_shape=[
            jax.ShapeDtypeStruct((R, HID), F32),
            jax.ShapeDtypeStruct((R, HID), F32),
        ],
        compiler_params=pltpu.CompilerParams(
            dimension_semantics=("arbitrary",)),
    )(h, Ws, Wd)


def _edge_step(e, g, W1e, b1, W2, b2, blk):
    R = e.shape[0]

    def body(e_ref, g_ref, w1_ref, b1_ref, w2_ref, b2_ref, o_ref):
        ev = e_ref[...].astype(F32)
        t = jnp.dot(ev, w1_ref[...], preferred_element_type=F32)
        t = jnp.maximum(t + g_ref[...].astype(F32) + b1_ref[...], 0.0)
        y = ev + jnp.dot(t, w2_ref[...],
                         preferred_element_type=F32) + b2_ref[...]
        o_ref[...] = y.astype(BF)

    return pl.pallas_call(
        body,
        grid=(R // blk,),
        in_specs=[
            pl.BlockSpec((blk, HID), lambda i: (i, 0)),
            pl.BlockSpec((blk, HID), lambda i: (i, 0)),
            pl.BlockSpec((HID, HID), lambda i: (0, 0)),
            pl.BlockSpec((1, HID), lambda i: (0, 0)),
            pl.BlockSpec((HID, HID), lambda i: (0, 0)),
            pl.BlockSpec((1, HID), lambda i: (0, 0)),
        ],
        out_specs=pl.BlockSpec((blk, HID), lambda i: (i, 0)),
        out_shape=jax.ShapeDtypeStruct((R, HID), BF),
        compiler_params=pltpu.CompilerParams(
            dimension_semantics=("arbitrary",)),
    )(e, g, W1e, b1.reshape(1, -1), W2, b2.reshape(1, -1))


def _node_step(h, acc, deg, Wh, Wa, b1, W2, b2, blk):
    R = h.shape[0]

    def body(h_ref, acc_ref, deg_ref, wh_ref, wa_ref, b1_ref, w2_ref,
             b2_ref, o_ref):
        d = deg_ref[0] + deg_ref[1]
        d = jnp.maximum(d[:, :1], 1.0)
        agg = (acc_ref[0] + acc_ref[1]) / d
        hv = h_ref[...]
        t = jnp.dot(hv, wh_ref[...], preferred_element_type=F32)
        t = t + jnp.dot(agg, wa_ref[...], preferred_element_type=F32)
        t = jnp.maximum(t + b1_ref[...], 0.0)
        o_ref[...] = hv + jnp.dot(t, w2_ref[...],
                                  preferred_element_type=F32) + b2_ref[...]

    return pl.pallas_call(
        body,
        grid=(R // blk,),
        in_specs=[
            pl.BlockSpec((blk, HID), lambda i: (i, 0)),
            pl.BlockSpec((2, blk, HID), lambda i: (0, i, 0)),
            pl.BlockSpec((2, blk, HID), lambda i: (0, i, 0)),
            pl.BlockSpec((HID, HID), lambda i: (0, 0)),
            pl.BlockSpec((HID, HID), lambda i: (0, 0)),
            pl.BlockSpec((1, HID), lambda i: (0, 0)),
            pl.BlockSpec((HID, HID), lambda i: (0, 0)),
            pl.BlockSpec((1, HID), lambda i: (0, 0)),
        ],
        out_specs=pl.BlockSpec((blk, HID), lambda i: (i, 0)),
        out_shape=jax.ShapeDtypeStruct((R, HID), F32),
        compiler_params=pltpu.CompilerParams(
            dimension_semantics=("arbitrary",)),
    )(h, acc, deg, Wh, Wa, b1.reshape(1, -1), W2, b2.reshape(1, -1))


def _sc_mesh():
    return plsc.VectorSubcoreMesh(core_axis_name="c", subcore_axis_name="s",
                                  num_cores=NC, num_subcores=NS)


@functools.cache
def _gather_kernel(E, NPAD):
    nchunk = E // CH
    niter = _cdiv(nchunk, NW)

    npairs = _cdiv(niter, 2)
    assert niter >= 2 and nchunk >= 2 * NW

    @functools.partial(
        pl.kernel,
        out_type=jax.ShapeDtypeStruct((E, HID // 2), I32),
        mesh=_sc_mesh(),
        scratch_types=[
            [pltpu.VMEM((CH,), jnp.int32)] * 2,
            [pltpu.VMEM((CH,), jnp.int32)] * 2,
            [pltpu.VMEM((CH, HID), F32)] * 2,
            [pltpu.VMEM((CH, HID), F32)] * 2,
            [pltpu.VMEM((CH, HID // 2), I32)] * 2,
            [pltpu.SemaphoreType.DMA] * 2,
            [pltpu.SemaphoreType.DMA] * 2,
            [pltpu.SemaphoreType.DMA] * 2,
        ],
        compiler_params=pltpu.CompilerParams(needs_layout_passes=False),
    )
    def kern(hs_hbm, hd_hbm, src_hbm, dst_hbm, g_hbm,
             si, di, sr, dr, gb, sem_i, sem_g, sem_w):
        wid = lax.axis_index("s") * NC + lax.axis_index("c")

        for b in range(2):
            cid = wid + NW * b
            base = cid * CH
            pltpu.async_copy(src_hbm.at[pl.ds(base, CH)], si[b], sem_i[b])
            pltpu.async_copy(dst_hbm.at[pl.ds(base, CH)], di[b], sem_i[b])

        @pl.loop(0, npairs)
        def _(g):
            for b in range(2):
                j = 2 * g + b
                cid = wid + NW * j

                @pl.when(cid < nchunk)
                def _():
                    @pl.when(g > 0)
                    def _():
                        pltpu.make_async_copy(
                            gb[b], g_hbm.at[pl.ds(0, CH)], sem_w[b]).wait()
                    pltpu.make_async_copy(
                        src_hbm.at[pl.ds(0, CH)], si[b], sem_i[b]).wait()
                    pltpu.make_async_copy(
                        dst_hbm.at[pl.ds(0, CH)], di[b], sem_i[b]).wait()
                    pltpu.async_copy(hs_hbm.at[si[b]], sr[b], sem_g[b])
                    pltpu.async_copy(hd_hbm.at[di[b]], dr[b], sem_g[b])

            for b in range(2):
                j = 2 * g + b
                cid = wid + NW * j
                base = cid * CH

                @pl.when(cid < nchunk)
                def _():
                    pltpu.make_async_copy(
                        hs_hbm.at[si[b]], sr[b], sem_g[b]).wait()
                    pltpu.make_async_copy(
                        hd_hbm.at[di[b]], dr[b], sem_g[b]).wait()

                    @pl.loop(0, CH, unroll=8)
                    def _(r):
                        for l in range(HID // 32):
                            a0 = (sr[b][r, pl.ds(32 * l, 16)]
                                  + dr[b][r, pl.ds(32 * l, 16)])
                            a1 = (sr[b][r, pl.ds(32 * l + 16, 16)]
                                  + dr[b][r, pl.ds(32 * l + 16, 16)])
                            pk = plsc.pack(
                                a0, a1, format=plsc.PackFormat.INTERLEAVED)
                            gb[b][r, pl.ds(16 * l, 16)] = plsc.bitcast(
                                pk, I32)

                    pltpu.async_copy(gb[b], g_hbm.at[pl.ds(base, CH)],
                                     sem_w[b])
                    ncid = cid + 2 * NW

                    @pl.when(ncid < nchunk)
                    def _():
                        nbase = ncid * CH
                        pltpu.async_copy(src_hbm.at[pl.ds(nbase, CH)],
                                         si[b], sem_i[b])
                        pltpu.async_copy(dst_hbm.at[pl.ds(nbase, CH)],
                                         di[b], sem_i[b])

        for b in range(2):
            pltpu.make_async_copy(gb[b], g_hbm.at[pl.ds(0, CH)],
                                  sem_w[b]).wait()

    return kern


@functools.cache
def _scatter_kernel(E, NPAD):
    CHS = 80
    nchunk = E // CHS
    assert nchunk * CHS == E
    niter = _cdiv(nchunk, NW)
    rz = NPAD // NS

    assert niter >= 2 and nchunk >= 2 * NW

    @functools.partial(
        pl.kernel,
        out_type=jax.ShapeDtypeStruct((NC, NPAD, HID), F32),
        mesh=_sc_mesh(),
        scratch_types=[
            [pltpu.VMEM((CHS,), jnp.int32)] * 2,
            pltpu.VMEM((CHS, HID // 2), I32),
            [pltpu.VMEM((CHS, HID), F32)] * 2,
            [pltpu.SemaphoreType.DMA] * 2,
            pltpu.SemaphoreType.DMA,
            [pltpu.SemaphoreType.DMA] * 2,
            pltpu.VMEM_SHARED((NPAD, HID), F32),
        ],
        compiler_params=pltpu.CompilerParams(needs_layout_passes=False),
    )
    def kern(e_hbm, dst_hbm, zeros_hbm, out_hbm, idx, rows, fr, sem_l,
             sem_r, sem_a, acc_sh):
        c = lax.axis_index("c")
        s = lax.axis_index("s")
        wid = s * NC + c
        base0 = wid * CHS
        pltpu.async_copy(dst_hbm.at[pl.ds(base0, CHS)], idx[0], sem_l[0])
        pltpu.async_copy(e_hbm.at[pl.ds(base0, CHS)], rows, sem_r)
        pltpu.sync_copy(zeros_hbm.at[pl.ds(s * rz, rz)],
                        acc_sh.at[pl.ds(s * rz, rz)])
        plsc.subcore_barrier()

        @pl.loop(0, niter)
        def _(i):
            cid = wid + NW * i
            b0 = lax.rem(i, 2)

            @pl.when(cid < nchunk)
            def _():
                for b in range(2):
                    @pl.when(b0 == b)
                    def _():
                        o = 1 - b
                        pltpu.make_async_copy(
                            dst_hbm.at[pl.ds(0, CHS)], idx[b],
                            sem_l[b]).wait()
                        pltpu.make_async_copy(
                            e_hbm.at[pl.ds(0, CHS)], rows, sem_r).wait()

                        @pl.loop(0, CHS, unroll=8)
                        def _(r):
                            for l in range(HID // 32):
                                bf = plsc.bitcast(
                                    rows[r, pl.ds(16 * l, 16)], BF)
                                u0, u1 = plsc.unpack(
                                    bf,
                                    format=plsc.PackFormat.INTERLEAVED)
                                fr[b][r, pl.ds(32 * l, 16)] = u0
                                fr[b][r, pl.ds(32 * l + 16, 16)] = u1

                        @pl.when(i > 0)
                        def _():
                            pltpu.make_async_copy(
                                fr[o], acc_sh.at[idx[o]], sem_a[o]).wait()
                        ncid = cid + NW

                        @pl.when(ncid < nchunk)
                        def _():
                            nbase = ncid * CHS
                            pltpu.async_copy(dst_hbm.at[pl.ds(nbase, CHS)],
                                             idx[o], sem_l[o])
                            pltpu.async_copy(e_hbm.at[pl.ds(nbase, CHS)],
                                             rows, sem_r)
                        pltpu.async_copy(fr[b], acc_sh.at[idx[b]],
                                         sem_a[b], add=True)

        last_i = (nchunk - 1 - wid) // NW
        bl = lax.rem(last_i, 2)
        for b in range(2):
            @pl.when(bl == b)
            def _():
                pltpu.make_async_copy(fr[b], acc_sh.at[idx[b]],
                                      sem_a[b]).wait()
        plsc.subcore_barrier()
        pltpu.sync_copy(acc_sh.at[pl.ds(s * rz, rz)],
                        out_hbm.at[c, pl.ds(s * rz, rz)])

    return kern


@functools.cache
def _deg_kernel(E, NPAD):
    nchunk = E // CH
    niter = _cdiv(nchunk, NW)
    rz = NPAD // NS

    @functools.partial(
        pl.kernel,
        out_type=jax.ShapeDtypeStruct((NC, NPAD, HID), F32),
        mesh=_sc_mesh(),
        scratch_types=[
            pltpu.VMEM((CH,), jnp.int32),
            pltpu.VMEM((CH, HID), F32),
            pltpu.VMEM_SHARED((NPAD, HID), F32),
        ],
    )
    def kern(dst_hbm, ones_hbm, zeros_hbm, out_hbm, idx_v, ones_v, acc_sh):
        c = lax.axis_index("c")
        s = lax.axis_index("s")
        wid = s * NC + c
        pltpu.sync_copy(ones_hbm, ones_v)
        pltpu.sync_copy(zeros_hbm.at[pl.ds(s * rz, rz)],
                        acc_sh.at[pl.ds(s * rz, rz)])
        plsc.subcore_barrier()

        @pl.loop(0, niter)
        def _(i):
            cid = wid + NW * i

            @pl.when(cid < nchunk)
            def _():
                pltpu.sync_copy(dst_hbm.at[pl.ds(cid * CH, CH)], idx_v)
                pltpu.sync_copy(ones_v, acc_sh.at[idx_v], add=True)

        plsc.subcore_barrier()
        pltpu.sync_copy(acc_sh.at[pl.ds(s * rz, rz)],
                        out_hbm.at[c, pl.ds(s * rz, rz)])

    return kern


def _bf_view(x32):
    E = x32.shape[0]
    return jax.lax.bitcast_convert_type(x32, BF).reshape(E, HID)


def _i32_view(xbf):
    E = xbf.shape[0]
    return jax.lax.bitcast_convert_type(
        xbf.reshape(E, HID // 2, 2), I32)


def _mgn_conv(p, h0, src, dst, e0, deg, NPAD, E):
    h = h0
    e = e0
    for st in p['steps']:
        (W1, b1), (W2, b2) = st['edge_mlp']
        W1e, W1s, W1d = W1[:HID], W1[HID:2 * HID], W1[2 * HID:]
        hs, hd = _hs_hd(h, W1s, W1d, NBLK)
        g32 = _gather_kernel(E, NPAD)(hs, hd, src, dst)
        e = _edge_step(e, _bf_view(g32), W1e[:, PI], b1[PI], W2[PI, :],
                       b2, EBLK)
        acc = _scatter_kernel(E, NPAD)(
            _i32_view(e), dst, jnp.zeros((NPAD, HID), F32))
        (Wn1, bn1), (Wn2, bn2) = st['node_mlp']
        Wh, Wa = Wn1[:HID], Wn1[HID:]
        h = _node_step(h, acc, deg, Wh, Wa[SIG], bn1, Wn2, bn2, NBLK)
    return h


def kernel(featr2, stmdist, edge_attr, edge_index, params):
    N = featr2.shape[0]
    E = edge_attr.shape[0]
    NPAD = _cdiv(N, NBLK) * NBLK

    r = [0, 0, 0, 1, 1, 2]
    c = [0, 1, 2, 1, 2, 2]
    t0 = featr2[:, 0][:, r, c]
    t1 = featr2[:, 1][:, r, c]
    t2 = featr2[:, 2].reshape(-1, 9)
    x = jnp.concatenate([t0, t1, t2, stmdist], axis=-1)
    x = jnp.pad(x, ((0, NPAD - N), (0, 0)))

    src = edge_index[0]
    dst = edge_index[1]
    ea = jnp.pad(edge_attr, ((0, 0), (0, 4)))

    deg = _deg_kernel(E, NPAD)(
        dst, jnp.ones((CH, HID), F32), jnp.zeros((NPAD, HID), F32))

    out = x
    nconv = len(params)
    for li, p in enumerate(params):
        (Wn1, bn1), (Wn2, bn2) = p['node_enc']
        if li == 0:
            h = _mlp2(out, Wn1, bn1, Wn2, bn2, NBLK, out_relu=False)
        else:
            h = _mlp2(out, Wn1, bn1, Wn2, bn2, NBLK, out_relu=False)
        (We1, be1), (We2, be2) = p['edge_enc']
        We1p = jnp.pad(We1, ((0, 4), (0, 0)))
        e = _mlp2(ea, We1p, be1, We2, be2, EBLK, out_relu=False,
                  out_dtype=BF)
        h = _mgn_conv(p, h, src, dst, e, deg, NPAD, E)
        (Wd1, bd1), (Wd2, bd2) = p['dec']
        out = _mlp2(h, Wd1, bd1, Wd2, bd2, NBLK, out_relu=(li < nconv - 1))
    return out[:N]


# speedup vs baseline: 4.3295x; 4.3295x over previous
import functools

import jax
import jax.numpy as jnp
from jax import lax
from jax.experimental import pallas as pl
from jax.experimental.pallas import tpu as pltpu
from jax.experimental.pallas import tpu_sc as plsc

F32 = jnp.float32
HID = 128
NC, NS = 2, 16
NW = NC * NS
CH = 128
NBLK = 1024
EBLK = 2560


def _cdiv(a, b):
    return (a + b - 1) // b


def _mlp2(x, W1, b1, W2, b2, blk, out_relu):
    R, cin = x.shape
    cout = W2.shape[1]

    def body(x_ref, w1_ref, b1_ref, w2_ref, b2_ref, o_ref):
        t = jnp.dot(x_ref[...], w1_ref[...], preferred_element_type=F32)
        t = jnp.maximum(t + b1_ref[...], 0.0)
        y = jnp.dot(t, w2_ref[...], preferred_element_type=F32) + b2_ref[...]
        if out_relu:
            y = jnp.maximum(y, 0.0)
        o_ref[...] = y

    return pl.pallas_call(
        body,
        grid=(R // blk,),
        in_specs=[
            pl.BlockSpec((blk, cin), lambda i: (i, 0)),
            pl.BlockSpec((cin, HID), lambda i: (0, 0)),
            pl.BlockSpec((1, HID), lambda i: (0, 0)),
            pl.BlockSpec((HID, cout), lambda i: (0, 0)),
            pl.BlockSpec((1, cout), lambda i: (0, 0)),
        ],
        out_specs=pl.BlockSpec((blk, cout), lambda i: (i, 0)),
        out_shape=jax.ShapeDtypeStruct((R, cout), F32),
        compiler_params=pltpu.CompilerParams(
            dimension_semantics=("arbitrary",)),
    )(x, W1, b1.reshape(1, -1), W2, b2.reshape(1, -1))


def _hs_hd(h, Ws, Wd, blk):
    R = h.shape[0]

    def body(h_ref, ws_ref, wd_ref, o1_ref, o2_ref):
        hv = h_ref[...]
        o1_ref[...] = jnp.dot(hv, ws_ref[...], preferred_element_type=F32)
        o2_ref[...] = jnp.dot(hv, wd_ref[...], preferred_element_type=F32)

    return pl.pallas_call(
        body,
        grid=(R // blk,),
        in_specs=[
            pl.BlockSpec((blk, HID), lambda i: (i, 0)),
            pl.BlockSpec((HID, HID), lambda i: (0, 0)),
            pl.BlockSpec((HID, HID), lambda i: (0, 0)),
        ],
        out_specs=[
            pl.BlockSpec((blk, HID), lambda i: (i, 0)),
            pl.BlockSpec((blk, HID), lambda i: (i, 0)),
        ],
        out_shape=[
            jax.ShapeDtypeStruct((R, HID), F32),
            jax.ShapeDtypeStruct((R, HID), F32),
        ],
        compiler_params=pltpu.CompilerParams(
            dimension_semantics=("arbitrary",)),
    )(h, Ws, Wd)


def _edge_step(e, g, W1e, b1, W2, b2, blk):
    R = e.shape[0]

    def body(e_ref, g_ref, w1_ref, b1_ref, w2_ref, b2_ref, o_ref):
        ev = e_ref[...]
        t = jnp.dot(ev, w1_ref[...], preferred_element_type=F32)
        t = jnp.maximum(t + g_ref[...] + b1_ref[...], 0.0)
        o_ref[...] = ev + jnp.dot(t, w2_ref[...],
                                  preferred_element_type=F32) + b2_ref[...]

    return pl.pallas_call(
        body,
        grid=(R // blk,),
        in_specs=[
            pl.BlockSpec((blk, HID), lambda i: (i, 0)),
            pl.BlockSpec((blk, HID), lambda i: (i, 0)),
            pl.BlockSpec((HID, HID), lambda i: (0, 0)),
            pl.BlockSpec((1, HID), lambda i: (0, 0)),
            pl.BlockSpec((HID, HID), lambda i: (0, 0)),
            pl.BlockSpec((1, HID), lambda i: (0, 0)),
        ],
        out_specs=pl.BlockSpec((blk, HID), lambda i: (i, 0)),
        out_shape=jax.ShapeDtypeStruct((R, HID), F32),
        compiler_params=pltpu.CompilerParams(
            dimension_semantics=("arbitrary",)),
    )(e, g, W1e, b1.reshape(1, -1), W2, b2.reshape(1, -1))


def _node_step(h, acc, deg, Wh, Wa, b1, W2, b2, blk):
    R = h.shape[0]

    def body(h_ref, acc_ref, deg_ref, wh_ref, wa_ref, b1_ref, w2_ref,
             b2_ref, o_ref):
        d = deg_ref[0] + deg_ref[1]
        d = jnp.maximum(d[:, :1], 1.0)
        agg = (acc_ref[0] + acc_ref[1]) / d
        hv = h_ref[...]
        t = jnp.dot(hv, wh_ref[...], preferred_element_type=F32)
        t = t + jnp.dot(agg, wa_ref[...], preferred_element_type=F32)
        t = jnp.maximum(t + b1_ref[...], 0.0)
        o_ref[...] = hv + jnp.dot(t, w2_ref[...],
                                  preferred_element_type=F32) + b2_ref[...]

    return pl.pallas_call(
        body,
        grid=(R // blk,),
        in_specs=[
            pl.BlockSpec((blk, HID), lambda i: (i, 0)),
            pl.BlockSpec((2, blk, HID), lambda i: (0, i, 0)),
            pl.BlockSpec((2, blk, HID), lambda i: (0, i, 0)),
            pl.BlockSpec((HID, HID), lambda i: (0, 0)),
            pl.BlockSpec((HID, HID), lambda i: (0, 0)),
            pl.BlockSpec((1, HID), lambda i: (0, 0)),
            pl.BlockSpec((HID, HID), lambda i: (0, 0)),
            pl.BlockSpec((1, HID), lambda i: (0, 0)),
        ],
        out_specs=pl.BlockSpec((blk, HID), lambda i: (i, 0)),
        out_shape=jax.ShapeDtypeStruct((R, HID), F32),
        compiler_params=pltpu.CompilerParams(
            dimension_semantics=("arbitrary",)),
    )(h, acc, deg, Wh, Wa, b1.reshape(1, -1), W2, b2.reshape(1, -1))


def _sc_mesh():
    return plsc.VectorSubcoreMesh(core_axis_name="c", subcore_axis_name="s",
                                  num_cores=NC, num_subcores=NS)


@functools.cache
def _gather_kernel(E, NPAD):
    nchunk = E // CH
    niter = _cdiv(nchunk, NW)

    npairs = _cdiv(niter, 2)
    assert niter >= 2 and nchunk >= 2 * NW

    @functools.partial(
        pl.kernel,
        out_type=jax.ShapeDtypeStruct((E, HID), F32),
        mesh=_sc_mesh(),
        scratch_types=[
            [pltpu.VMEM((CH,), jnp.int32)] * 2,
            [pltpu.VMEM((CH,), jnp.int32)] * 2,
            [pltpu.VMEM((CH, HID), F32)] * 2,
            [pltpu.VMEM((CH, HID), F32)] * 2,
            [pltpu.SemaphoreType.DMA] * 2,
            [pltpu.SemaphoreType.DMA] * 2,
            [pltpu.SemaphoreType.DMA] * 2,
        ],
    )
    def kern(hs_hbm, hd_hbm, src_hbm, dst_hbm, g_hbm,
             si, di, sr, dr, sem_i, sem_g, sem_w):
        wid = lax.axis_index("s") * NC + lax.axis_index("c")

        for b in range(2):
            cid = wid + NW * b
            base = cid * CH
            pltpu.async_copy(src_hbm.at[pl.ds(base, CH)], si[b], sem_i[b])
            pltpu.async_copy(dst_hbm.at[pl.ds(base, CH)], di[b], sem_i[b])

        @pl.loop(0, npairs)
        def _(g):
            for b in range(2):
                j = 2 * g + b
                cid = wid + NW * j

                @pl.when(cid < nchunk)
                def _():
                    @pl.when(g > 0)
                    def _():
                        pltpu.make_async_copy(
                            sr[b], g_hbm.at[pl.ds(0, CH)], sem_w[b]).wait()
                    pltpu.make_async_copy(
                        src_hbm.at[pl.ds(0, CH)], si[b], sem_i[b]).wait()
                    pltpu.make_async_copy(
                        dst_hbm.at[pl.ds(0, CH)], di[b], sem_i[b]).wait()
                    pltpu.async_copy(hs_hbm.at[si[b]], sr[b], sem_g[b])
                    pltpu.async_copy(hd_hbm.at[di[b]], dr[b], sem_g[b])

            for b in range(2):
                j = 2 * g + b
                cid = wid + NW * j
                base = cid * CH

                @pl.when(cid < nchunk)
                def _():
                    pltpu.make_async_copy(
                        hs_hbm.at[si[b]], sr[b], sem_g[b]).wait()
                    pltpu.make_async_copy(
                        hd_hbm.at[di[b]], dr[b], sem_g[b]).wait()

                    @pl.loop(0, CH, unroll=8)
                    def _(r):
                        for l in range(HID // 16):
                            sl = pl.ds(16 * l, 16)
                            plsc.addupdate(sr[b].at[r, sl], dr[b][r, sl])

                    pltpu.async_copy(sr[b], g_hbm.at[pl.ds(base, CH)],
                                     sem_w[b])
                    ncid = cid + 2 * NW

                    @pl.when(ncid < nchunk)
                    def _():
                        nbase = ncid * CH
                        pltpu.async_copy(src_hbm.at[pl.ds(nbase, CH)],
                                         si[b], sem_i[b])
                        pltpu.async_copy(dst_hbm.at[pl.ds(nbase, CH)],
                                         di[b], sem_i[b])

        for b in range(2):
            pltpu.make_async_copy(sr[b], g_hbm.at[pl.ds(0, CH)],
                                  sem_w[b]).wait()

    return kern


@functools.cache
def _scatter_kernel(E, NPAD):
    nchunk = E // CH
    niter = _cdiv(nchunk, NW)
    rz = NPAD // NS

    assert niter >= 2 and nchunk >= 2 * NW

    @functools.partial(
        pl.kernel,
        out_type=jax.ShapeDtypeStruct((NC, NPAD, HID), F32),
        mesh=_sc_mesh(),
        scratch_types=[
            [pltpu.VMEM((CH,), jnp.int32)] * 2,
            [pltpu.VMEM((CH, HID), F32)] * 2,
            [pltpu.SemaphoreType.DMA] * 2,
            pltpu.VMEM_SHARED((NPAD, HID), F32),
        ],
    )
    def kern(e_hbm, dst_hbm, zeros_hbm, out_hbm, idx, rows, sem_l, acc_sh):
        c = lax.axis_index("c")
        s = lax.axis_index("s")
        wid = s * NC + c
        for b in range(2):
            base = (wid + NW * b) * CH
            pltpu.async_copy(dst_hbm.at[pl.ds(base, CH)], idx[b], sem_l[b])
            pltpu.async_copy(e_hbm.at[pl.ds(base, CH)], rows[b], sem_l[b])
        pltpu.sync_copy(zeros_hbm.at[pl.ds(s * rz, rz)],
                        acc_sh.at[pl.ds(s * rz, rz)])
        plsc.subcore_barrier()

        @pl.loop(0, niter)
        def _(i):
            cid = wid + NW * i
            b0 = lax.rem(i, 2)

            @pl.when(cid < nchunk)
            def _():
                for b in range(2):
                    @pl.when(b0 == b)
                    def _():
                        pltpu.make_async_copy(
                            dst_hbm.at[pl.ds(0, CH)], idx[b],
                            sem_l[b]).wait()
                        pltpu.make_async_copy(
                            e_hbm.at[pl.ds(0, CH)], rows[b],
                            sem_l[b]).wait()
                        pltpu.sync_copy(rows[b], acc_sh.at[idx[b]],
                                        add=True)
                        ncid = cid + 2 * NW

                        @pl.when(ncid < nchunk)
                        def _():
                            nbase = ncid * CH
                            pltpu.async_copy(dst_hbm.at[pl.ds(nbase, CH)],
                                             idx[b], sem_l[b])
                            pltpu.async_copy(e_hbm.at[pl.ds(nbase, CH)],
                                             rows[b], sem_l[b])

        plsc.subcore_barrier()
        pltpu.sync_copy(acc_sh.at[pl.ds(s * rz, rz)],
                        out_hbm.at[c, pl.ds(s * rz, rz)])

    return kern


@functools.cache
def _deg_kernel(E, NPAD):
    nchunk = E // CH
    niter = _cdiv(nchunk, NW)
    rz = NPAD // NS

    @functools.partial(
        pl.kernel,
        out_type=jax.ShapeDtypeStruct((NC, NPAD, HID), F32),
        mesh=_sc_mesh(),
        scratch_types=[
            pltpu.VMEM((CH,), jnp.int32),
            pltpu.VMEM((CH, HID), F32),
            pltpu.VMEM_SHARED((NPAD, HID), F32),
        ],
    )
    def kern(dst_hbm, ones_hbm, zeros_hbm, out_hbm, idx_v, ones_v, acc_sh):
        c = lax.axis_index("c")
        s = lax.axis_index("s")
        wid = s * NC + c
        pltpu.sync_copy(ones_hbm, ones_v)
        pltpu.sync_copy(zeros_hbm.at[pl.ds(s * rz, rz)],
                        acc_sh.at[pl.ds(s * rz, rz)])
        plsc.subcore_barrier()

        @pl.loop(0, niter)
        def _(i):
            cid = wid + NW * i

            @pl.when(cid < nchunk)
            def _():
                pltpu.sync_copy(dst_hbm.at[pl.ds(cid * CH, CH)], idx_v)
                pltpu.sync_copy(ones_v, acc_sh.at[idx_v], add=True)

        plsc.subcore_barrier()
        pltpu.sync_copy(acc_sh.at[pl.ds(s * rz, rz)],
                        out_hbm.at[c, pl.ds(s * rz, rz)])

    return kern


def _mgn_conv(p, h0, src, dst, e0, deg, NPAD, E):
    h = h0
    e = e0
    for st in p['steps']:
        (W1, b1), (W2, b2) = st['edge_mlp']
        W1e, W1s, W1d = W1[:HID], W1[HID:2 * HID], W1[2 * HID:]
        hs, hd = _hs_hd(h, W1s, W1d, NBLK)
        g = _gather_kernel(E, NPAD)(hs, hd, src, dst)
        e = _edge_step(e, g, W1e, b1, W2, b2, EBLK)
        acc = _scatter_kernel(E, NPAD)(
            e, dst, jnp.zeros((NPAD, HID), F32))
        (Wn1, bn1), (Wn2, bn2) = st['node_mlp']
        Wh, Wa = Wn1[:HID], Wn1[HID:]
        h = _node_step(h, acc, deg, Wh, Wa, bn1, Wn2, bn2, NBLK)
    return h


def kernel(featr2, stmdist, edge_attr, edge_index, params):
    N = featr2.shape[0]
    E = edge_attr.shape[0]
    NPAD = _cdiv(N, NBLK) * NBLK

    r = [0, 0, 0, 1, 1, 2]
    c = [0, 1, 2, 1, 2, 2]
    t0 = featr2[:, 0][:, r, c]
    t1 = featr2[:, 1][:, r, c]
    t2 = featr2[:, 2].reshape(-1, 9)
    x = jnp.concatenate([t0, t1, t2, stmdist], axis=-1)
    x = jnp.pad(x, ((0, NPAD - N), (0, 0)))

    src = edge_index[0]
    dst = edge_index[1]
    ea = jnp.pad(edge_attr, ((0, 0), (0, 4)))

    deg = _deg_kernel(E, NPAD)(
        dst, jnp.ones((CH, HID), F32), jnp.zeros((NPAD, HID), F32))

    out = x
    nconv = len(params)
    for li, p in enumerate(params):
        (Wn1, bn1), (Wn2, bn2) = p['node_enc']
        if li == 0:
            h = _mlp2(out, Wn1, bn1, Wn2, bn2, NBLK, out_relu=False)
        else:
            h = _mlp2(out, Wn1, bn1, Wn2, bn2, NBLK, out_relu=False)
        (We1, be1), (We2, be2) = p['edge_enc']
        We1p = jnp.pad(We1, ((0, 4), (0, 0)))
        e = _mlp2(ea, We1p, be1, We2, be2, EBLK, out_relu=False)
        h = _mgn_conv(p, h, src, dst, e, deg, NPAD, E)
        (Wd1, bd1), (Wd2, bd2) = p['dec']
        out = _mlp2(h, Wd1, bd1, Wd2, bd2, NBLK, out_relu=(li < nconv - 1))
    return out[:N]


# speedup vs baseline: 4.3344x; 1.0011x over previous
import functools

import jax
import jax.numpy as jnp
from jax import lax
from jax.experimental import pallas as pl
from jax.experimental.pallas import tpu as pltpu
from jax.experimental.pallas import tpu_sc as plsc

F32 = jnp.float32
HID = 128
NC, NS = 2, 16
NW = NC * NS
CH = 128
NBLK = 1024
EBLK = 2560


def _cdiv(a, b):
    return (a + b - 1) // b


def _mlp2(x, W1, b1, W2, b2, blk, out_relu):
    R, cin = x.shape
    cout = W2.shape[1]

    def body(x_ref, w1_ref, b1_ref, w2_ref, b2_ref, o_ref):
        t = jnp.dot(x_ref[...], w1_ref[...], preferred_element_type=F32)
        t = jnp.maximum(t + b1_ref[...], 0.0)
        y = jnp.dot(t, w2_ref[...], preferred_element_type=F32) + b2_ref[...]
        if out_relu:
            y = jnp.maximum(y, 0.0)
        o_ref[...] = y

    return pl.pallas_call(
        body,
        grid=(R // blk,),
        in_specs=[
            pl.BlockSpec((blk, cin), lambda i: (i, 0)),
            pl.BlockSpec((cin, HID), lambda i: (0, 0)),
            pl.BlockSpec((1, HID), lambda i: (0, 0)),
            pl.BlockSpec((HID, cout), lambda i: (0, 0)),
            pl.BlockSpec((1, cout), lambda i: (0, 0)),
        ],
        out_specs=pl.BlockSpec((blk, cout), lambda i: (i, 0)),
        out_shape=jax.ShapeDtypeStruct((R, cout), F32),
        compiler_params=pltpu.CompilerParams(
            dimension_semantics=("arbitrary",)),
    )(x, W1, b1.reshape(1, -1), W2, b2.reshape(1, -1))


def _hs_hd(h, Ws, Wd, blk):
    R = h.shape[0]

    def body(h_ref, ws_ref, wd_ref, o1_ref, o2_ref):
        hv = h_ref[...]
        o1_ref[...] = jnp.dot(hv, ws_ref[...], preferred_element_type=F32)
        o2_ref[...] = jnp.dot(hv, wd_ref[...], preferred_element_type=F32)

    return pl.pallas_call(
        body,
        grid=(R // blk,),
        in_specs=[
            pl.BlockSpec((blk, HID), lambda i: (i, 0)),
            pl.BlockSpec((HID, HID), lambda i: (0, 0)),
            pl.BlockSpec((HID, HID), lambda i: (0, 0)),
        ],
        out_specs=[
            pl.BlockSpec((blk, HID), lambda i: (i, 0)),
            pl.BlockSpec((blk, HID), lambda i: (i, 0)),
        ],
        out_shape=[
            jax.ShapeDtypeStruct((R, HID), F32),
            jax.ShapeDtypeStruct((R, HID), F32),
        ],
        compiler_params=pltpu.CompilerParams(
            dimension_semantics=("arbitrary",)),
    )(h, Ws, Wd)


def _edge_step(e, g, W1e, b1, W2, b2, blk):
    R = e.shape[0]

    def body(e_ref, g_ref, w1_ref, b1_ref, w2_ref, b2_ref, o_ref):
        ev = e_ref[...]
        t = jnp.dot(ev, w1_ref[...], preferred_element_type=F32)
        t = jnp.maximum(t + g_ref[...] + b1_ref[...], 0.0)
        o_ref[...] = ev + jnp.dot(t, w2_ref[...],
                                  preferred_element_type=F32) + b2_ref[...]

    return pl.pallas_call(
        body,
        grid=(R // blk,),
        in_specs=[
            pl.BlockSpec((blk, HID), lambda i: (i, 0)),
            pl.BlockSpec((blk, HID), lambda i: (i, 0)),
            pl.BlockSpec((HID, HID), lambda i: (0, 0)),
            pl.BlockSpec((1, HID), lambda i: (0, 0)),
            pl.BlockSpec((HID, HID), lambda i: (0, 0)),
            pl.BlockSpec((1, HID), lambda i: (0, 0)),
        ],
        out_specs=pl.BlockSpec((blk, HID), lambda i: (i, 0)),
        out_shape=jax.ShapeDtypeStruct((R, HID), F32),
        compiler_params=pltpu.CompilerParams(
            dimension_semantics=("arbitrary",)),
    )(e, g, W1e, b1.reshape(1, -1), W2, b2.reshape(1, -1))


def _node_step(h, acc, deg, Wh, Wa, b1, W2, b2, blk):
    R = h.shape[0]

    def body(h_ref, acc_ref, deg_ref, wh_ref, wa_ref, b1_ref, w2_ref,
             b2_ref, o_ref):
        d = deg_ref[0] + deg_ref[1]
        d = jnp.maximum(d[:, :1], 1.0)
        agg = (acc_ref[0] + acc_ref[1]) / d
        hv = h_ref[...]
        t = jnp.dot(hv, wh_ref[...], preferred_element_type=F32)
        t = t + jnp.dot(agg, wa_ref[...], preferred_element_type=F32)
        t = jnp.maximum(t + b1_ref[...], 0.0)
        o_ref[...] = hv + jnp.dot(t, w2_ref[...],
                                  preferred_element_type=F32) + b2_ref[...]

    return pl.pallas_call(
        body,
        grid=(R // blk,),
        in_specs=[
            pl.BlockSpec((blk, HID), lambda i: (i, 0)),
            pl.BlockSpec((2, blk, HID), lambda i: (0, i, 0)),
            pl.BlockSpec((2, blk, HID), lambda i: (0, i, 0)),
            pl.BlockSpec((HID, HID), lambda i: (0, 0)),
            pl.BlockSpec((HID, HID), lambda i: (0, 0)),
            pl.BlockSpec((1, HID), lambda i: (0, 0)),
            pl.BlockSpec((HID, HID), lambda i: (0, 0)),
            pl.BlockSpec((1, HID), lambda i: (0, 0)),
        ],
        out_specs=pl.BlockSpec((blk, HID), lambda i: (i, 0)),
        out_shape=jax.ShapeDtypeStruct((R, HID), F32),
        compiler_params=pltpu.CompilerParams(
            dimension_semantics=("arbitrary",)),
    )(h, acc, deg, Wh, Wa, b1.reshape(1, -1), W2, b2.reshape(1, -1))


def _sc_mesh():
    return plsc.VectorSubcoreMesh(core_axis_name="c", subcore_axis_name="s",
                                  num_cores=NC, num_subcores=NS)


@functools.cache
def _gather_kernel(E, NPAD):
    nchunk = E // CH
    niter = _cdiv(nchunk, NW)

    npairs = _cdiv(niter, 2)
    assert niter >= 2 and nchunk >= 2 * NW

    @functools.partial(
        pl.kernel,
        out_type=jax.ShapeDtypeStruct((E, HID), F32),
        mesh=_sc_mesh(),
        scratch_types=[
            [pltpu.VMEM((CH,), jnp.int32)] * 2,
            [pltpu.VMEM((CH,), jnp.int32)] * 2,
            [pltpu.VMEM((CH, HID), F32)] * 2,
            [pltpu.VMEM((CH, HID), F32)] * 2,
            [pltpu.SemaphoreType.DMA] * 2,
            [pltpu.SemaphoreType.DMA] * 2,
            [pltpu.SemaphoreType.DMA] * 2,
        ],
    )
    def kern(hs_hbm, hd_hbm, src_hbm, dst_hbm, g_hbm,
             si, di, sr, dr, sem_i, sem_g, sem_w):
        wid = lax.axis_index("s") * NC + lax.axis_index("c")

        for b in range(2):
            cid = wid + NW * b
            base = cid * CH
            pltpu.async_copy(src_hbm.at[pl.ds(base, CH)], si[b], sem_i[b])
            pltpu.async_copy(dst_hbm.at[pl.ds(base, CH)], di[b], sem_i[b])

        @pl.loop(0, npairs)
        def _(g):
            for b in range(2):
                j = 2 * g + b
                cid = wid + NW * j

                @pl.when(cid < nchunk)
                def _():
                    @pl.when(g > 0)
                    def _():
                        pltpu.make_async_copy(
                            sr[b], g_hbm.at[pl.ds(0, CH)], sem_w[b]).wait()
                    pltpu.make_async_copy(
                        src_hbm.at[pl.ds(0, CH)], si[b], sem_i[b]).wait()
                    pltpu.make_async_copy(
                        dst_hbm.at[pl.ds(0, CH)], di[b], sem_i[b]).wait()
                    pltpu.async_copy(hs_hbm.at[si[b]], sr[b], sem_g[b])
                    pltpu.async_copy(hd_hbm.at[di[b]], dr[b], sem_g[b])

            for b in range(2):
                j = 2 * g + b
                cid = wid + NW * j
                base = cid * CH

                @pl.when(cid < nchunk)
                def _():
                    pltpu.make_async_copy(
                        hs_hbm.at[si[b]], sr[b], sem_g[b]).wait()
                    pltpu.make_async_copy(
                        hd_hbm.at[di[b]], dr[b], sem_g[b]).wait()

                    @pl.loop(0, CH, unroll=8)
                    def _(r):
                        for l in range(HID // 16):
                            sl = pl.ds(16 * l, 16)
                            plsc.addupdate(sr[b].at[r, sl], dr[b][r, sl])

                    pltpu.async_copy(sr[b], g_hbm.at[pl.ds(base, CH)],
                                     sem_w[b])
                    ncid = cid + 2 * NW

                    @pl.when(ncid < nchunk)
                    def _():
                        nbase = ncid * CH
                        pltpu.async_copy(src_hbm.at[pl.ds(nbase, CH)],
                                         si[b], sem_i[b])
                        pltpu.async_copy(dst_hbm.at[pl.ds(nbase, CH)],
                                         di[b], sem_i[b])

        for b in range(2):
            pltpu.make_async_copy(sr[b], g_hbm.at[pl.ds(0, CH)],
                                  sem_w[b]).wait()

    return kern


@functools.cache
def _scatter_kernel(E, NPAD):
    nchunk = E // CH
    niter = _cdiv(nchunk, NW)
    rz = NPAD // NS

    assert niter >= 2 and nchunk >= 2 * NW

    @functools.partial(
        pl.kernel,
        out_type=jax.ShapeDtypeStruct((NC, NPAD, HID), F32),
        mesh=_sc_mesh(),
        scratch_types=[
            [pltpu.VMEM((CH,), jnp.int32)] * 2,
            [pltpu.VMEM((CH, HID), F32)] * 2,
            [pltpu.SemaphoreType.DMA] * 2,
            [pltpu.SemaphoreType.DMA] * 2,
            pltpu.VMEM_SHARED((NPAD, HID), F32),
        ],
    )
    def kern(e_hbm, dst_hbm, zeros_hbm, out_hbm, idx, rows, sem_i, sem_e,
             acc_sh):
        c = lax.axis_index("c")
        s = lax.axis_index("s")
        wid = s * NC + c
        for b in range(2):
            base = (wid + NW * b) * CH
            pltpu.async_copy(dst_hbm.at[pl.ds(base, CH)], idx[b], sem_i[b])
            pltpu.async_copy(e_hbm.at[pl.ds(base, CH)], rows[b], sem_e[b])
        pltpu.sync_copy(zeros_hbm.at[pl.ds(s * rz, rz)],
                        acc_sh.at[pl.ds(s * rz, rz)])
        plsc.subcore_barrier()

        @pl.loop(0, niter)
        def _(i):
            cid = wid + NW * i
            b0 = lax.rem(i, 2)

            @pl.when(cid < nchunk)
            def _():
                for b in range(2):
                    @pl.when(b0 == b)
                    def _():
                        pltpu.make_async_copy(
                            dst_hbm.at[pl.ds(0, CH)], idx[b],
                            sem_i[b]).wait()
                        pltpu.make_async_copy(
                            e_hbm.at[pl.ds(0, CH)], rows[b],
                            sem_e[b]).wait()
                        pltpu.sync_copy(rows[b], acc_sh.at[idx[b]],
                                        add=True)
                        ncid = cid + 2 * NW

                        @pl.when(ncid < nchunk)
                        def _():
                            nbase = ncid * CH
                            pltpu.async_copy(dst_hbm.at[pl.ds(nbase, CH)],
                                             idx[b], sem_i[b])
                            pltpu.async_copy(e_hbm.at[pl.ds(nbase, CH)],
                                             rows[b], sem_e[b])

        plsc.subcore_barrier()
        pltpu.sync_copy(acc_sh.at[pl.ds(s * rz, rz)],
                        out_hbm.at[c, pl.ds(s * rz, rz)])

    return kern


@functools.cache
def _deg_kernel(E, NPAD):
    nchunk = E // CH
    niter = _cdiv(nchunk, NW)
    rz = NPAD // NS

    @functools.partial(
        pl.kernel,
        out_type=jax.ShapeDtypeStruct((NC, NPAD, HID), F32),
        mesh=_sc_mesh(),
        scratch_types=[
            pltpu.VMEM((CH,), jnp.int32),
            pltpu.VMEM((CH, HID), F32),
            pltpu.VMEM_SHARED((NPAD, HID), F32),
        ],
    )
    def kern(dst_hbm, ones_hbm, zeros_hbm, out_hbm, idx_v, ones_v, acc_sh):
        c = lax.axis_index("c")
        s = lax.axis_index("s")
        wid = s * NC + c
        pltpu.sync_copy(ones_hbm, ones_v)
        pltpu.sync_copy(zeros_hbm.at[pl.ds(s * rz, rz)],
                        acc_sh.at[pl.ds(s * rz, rz)])
        plsc.subcore_barrier()

        @pl.loop(0, niter)
        def _(i):
            cid = wid + NW * i

            @pl.when(cid < nchunk)
            def _():
                pltpu.sync_copy(dst_hbm.at[pl.ds(cid * CH, CH)], idx_v)
                pltpu.sync_copy(ones_v, acc_sh.at[idx_v], add=True)

        plsc.subcore_barrier()
        pltpu.sync_copy(acc_sh.at[pl.ds(s * rz, rz)],
                        out_hbm.at[c, pl.ds(s * rz, rz)])

    return kern


def _mgn_conv(p, h0, src, dst, e0, deg, NPAD, E):
    h = h0
    e = e0
    for st in p['steps']:
        (W1, b1), (W2, b2) = st['edge_mlp']
        W1e, W1s, W1d = W1[:HID], W1[HID:2 * HID], W1[2 * HID:]
        hs, hd = _hs_hd(h, W1s, W1d, NBLK)
        g = _gather_kernel(E, NPAD)(hs, hd, src, dst)
        e = _edge_step(e, g, W1e, b1, W2, b2, EBLK)
        acc = _scatter_kernel(E, NPAD)(
            e, dst, jnp.zeros((NPAD, HID), F32))
        (Wn1, bn1), (Wn2, bn2) = st['node_mlp']
        Wh, Wa = Wn1[:HID], Wn1[HID:]
        h = _node_step(h, acc, deg, Wh, Wa, bn1, Wn2, bn2, NBLK)
    return h


def kernel(featr2, stmdist, edge_attr, edge_index, params):
    N = featr2.shape[0]
    E = edge_attr.shape[0]
    NPAD = _cdiv(N, NBLK) * NBLK

    r = [0, 0, 0, 1, 1, 2]
    c = [0, 1, 2, 1, 2, 2]
    t0 = featr2[:, 0][:, r, c]
    t1 = featr2[:, 1][:, r, c]
    t2 = featr2[:, 2].reshape(-1, 9)
    x = jnp.concatenate([t0, t1, t2, stmdist], axis=-1)
    x = jnp.pad(x, ((0, NPAD - N), (0, 0)))

    src = edge_index[0]
    dst = edge_index[1]
    ea = jnp.pad(edge_attr, ((0, 0), (0, 4)))

    deg = _deg_kernel(E, NPAD)(
        dst, jnp.ones((CH, HID), F32), jnp.zeros((NPAD, HID), F32))

    out = x
    nconv = len(params)
    for li, p in enumerate(params):
        (Wn1, bn1), (Wn2, bn2) = p['node_enc']
        if li == 0:
            h = _mlp2(out, Wn1, bn1, Wn2, bn2, NBLK, out_relu=False)
        else:
            h = _mlp2(out, Wn1, bn1, Wn2, bn2, NBLK, out_relu=False)
        (We1, be1), (We2, be2) = p['edge_enc']
        We1p = jnp.pad(We1, ((0, 4), (0, 0)))
        e = _mlp2(ea, We1p, be1, We2, be2, EBLK, out_relu=False)
        h = _mgn_conv(p, h, src, dst, e, deg, NPAD, E)
        (Wd1, bd1), (Wd2, bd2) = p['dec']
        out = _mlp2(h, Wd1, bd1, Wd2, bd2, NBLK, out_relu=(li < nconv - 1))
    return out[:N]
